# baseline (speedup 1.0000x reference)
import jax
import jax.numpy as jnp
from jax import lax
from jax.experimental import pallas as pl
from jax.experimental.pallas import tpu as pltpu
from jax.experimental.pallas import tpu_sc as plsc

_VOCAB = 1000000
_EMBED = 64
_BATCH = 4096
_SEQ = 200
_HALF = _EMBED // 2

_NC = 2
_NS = 16
_NW = _NC * _NS

_TOTAL = _BATCH * _SEQ
_BPW = _BATCH // _NW
_RPC = 2
_CROWS = _RPC * _SEQ
_IW = 100
_IPC = _CROWS // _IW
_NBUF = 4
_KMAX = _BPW // (_RPC * _NBUF)


def _sc_body(x2, sincos, table, out,
             idx_v, sc_v,
             r0, r1, r2, r3,
             si0, si1, si2, si3,
             so0, so1, so2, so3):
    rows = [r0, r1, r2, r3]
    sin_ = [si0, si1, si2, si3]
    sout = [so0, so1, so2, so3]
    wid = lax.axis_index("s") * _NC + lax.axis_index("c")

    pltpu.sync_copy(sincos, sc_v)

    def block(k, carry):
        brow0 = wid * _BPW + _RPC * _NBUF * k
        pltpu.sync_copy(x2.at[pl.ds(brow0 * _RPC, _IPC * _NBUF)], idx_v)

        hin = []
        for b in range(_NBUF):
            for j in range(_IPC):
                hin.append(pltpu.async_copy(
                    table.at[idx_v.at[_IPC * b + j]],
                    rows[b].at[pl.ds(j * _IW, _IW)],
                    sin_[b],
                ))

        hout = []
        for b in range(_NBUF):
            for j in range(_IPC):
                hin[_IPC * b + j].wait()

            rb = rows[b]

            @plsc.parallel_loop(0, _SEQ, unroll=2)
            def _(t):
                c0 = sc_v[t, 0:16]
                c1 = sc_v[t, 16:32]
                s0 = sc_v[t, 32:48]
                s1 = sc_v[t, 48:64]
                for s in range(_RPC):
                    r = s * _SEQ + t
                    e0 = rb[r, 0:16]
                    e1 = rb[r, 16:32]
                    o0 = rb[r, 32:48]
                    o1 = rb[r, 48:64]
                    rb[r, 0:16] = e0 * c0 - o0 * s0
                    rb[r, 16:32] = e1 * c1 - o1 * s1
                    rb[r, 32:48] = e0 * s0 + o0 * c0
                    rb[r, 48:64] = e1 * s1 + o1 * c1

            orow = (brow0 + _RPC * b) * _SEQ
            hout.append(pltpu.async_copy(
                rb, out.at[pl.ds(orow, _CROWS)], sout[b]))
        for h in hout:
            h.wait()
        return carry

    lax.fori_loop(0, _KMAX, block, 0)


@jax.jit
def _sc_call(x2, sincos, table):
    mesh = plsc.VectorSubcoreMesh(core_axis_name="c", subcore_axis_name="s")
    f = pl.kernel(
        _sc_body,
        mesh=mesh,
        compiler_params=pltpu.CompilerParams(use_tc_tiling_on_sc=False),
        out_type=jax.ShapeDtypeStruct((_TOTAL, _EMBED), jnp.float32),
        scratch_types=[
            pltpu.VMEM((_IPC * _NBUF, _IW), jnp.int32),
            pltpu.VMEM((_SEQ, _EMBED), jnp.float32),
        ] + [pltpu.VMEM((_CROWS, _EMBED), jnp.float32)] * _NBUF
          + [pltpu.SemaphoreType.DMA] * (2 * _NBUF),
    )
    return f(x2, sincos, table)


def kernel(x, table):
    if x.ndim == 1:
        x = x[None, :]
    x2 = x.astype(jnp.int32).reshape(_TOTAL // _IW, _IW)
    freqs = 1.0 / (10000.0 ** (jnp.arange(_HALF, dtype=jnp.float32) / _EMBED))
    ang = jnp.arange(_SEQ, dtype=jnp.float32)[:, None] * freqs[None, :]
    sincos = jnp.concatenate([jnp.cos(ang), jnp.sin(ang)], axis=-1)
    out = _sc_call(x2, sincos, table)
    return out.reshape(_BATCH, _SEQ, _EMBED)


# speedup vs baseline: 1.0039x; 1.0039x over previous
import jax
import jax.numpy as jnp
from jax import lax
from jax.experimental import pallas as pl
from jax.experimental.pallas import tpu as pltpu
from jax.experimental.pallas import tpu_sc as plsc

_VOCAB = 1000000
_EMBED = 64
_BATCH = 4096
_SEQ = 200
_HALF = _EMBED // 2

_NC = 2
_NS = 16
_NW = _NC * _NS

_BPW = _BATCH // _NW
_RPC = 2
_SPLITS = ((0, 104), (104, 96))
_NBUF = 4
_ROWS_PER_IT = _RPC * _NBUF
_KMAX = _BPW // _ROWS_PER_IT


def _sc_body(x, sincos, table, out,
             idx_v, sc_v,
             r0, r1, r2, r3,
             si0, si1, si2, si3,
             so0, so1, so2, so3):
    rows = [r0, r1, r2, r3]
    sin_ = [si0, si1, si2, si3]
    sout = [so0, so1, so2, so3]
    wid = lax.axis_index("s") * _NC + lax.axis_index("c")

    pltpu.sync_copy(sincos, sc_v)

    def block(k, carry):
        brow0 = wid * _BPW + _ROWS_PER_IT * k
        pltpu.sync_copy(x.at[pl.ds(brow0, _ROWS_PER_IT)], idx_v)

        hin = []
        for b in range(_NBUF):
            for s in range(_RPC):
                for (off, width) in _SPLITS:
                    hin.append(pltpu.async_copy(
                        table.at[idx_v.at[_RPC * b + s, pl.ds(off, width)]],
                        rows[b].at[s, pl.ds(off, width)],
                        sin_[b],
                    ))

        npc = _RPC * len(_SPLITS)
        hout = []
        for b in range(_NBUF):
            for h in hin[npc * b:npc * (b + 1)]:
                h.wait()

            rb = rows[b]

            @plsc.parallel_loop(0, _SEQ, unroll=2)
            def _(t):
                c0 = sc_v[t, 0:16]
                c1 = sc_v[t, 16:32]
                s0 = sc_v[t, 32:48]
                s1 = sc_v[t, 48:64]
                for s in range(_RPC):
                    e0 = rb[s, t, 0:16]
                    e1 = rb[s, t, 16:32]
                    o0 = rb[s, t, 32:48]
                    o1 = rb[s, t, 48:64]
                    rb[s, t, 0:16] = e0 * c0 - o0 * s0
                    rb[s, t, 16:32] = e1 * c1 - o1 * s1
                    rb[s, t, 32:48] = e0 * s0 + o0 * c0
                    rb[s, t, 48:64] = e1 * s1 + o1 * c1

            hout.append(pltpu.async_copy(
                rb, out.at[pl.ds(brow0 + _RPC * b, _RPC)], sout[b]))
        for h in hout:
            h.wait()
        return carry

    lax.fori_loop(0, _KMAX, block, 0)


@jax.jit
def _sc_call(x, sincos, table):
    mesh = plsc.VectorSubcoreMesh(core_axis_name="c", subcore_axis_name="s")
    f = pl.kernel(
        _sc_body,
        mesh=mesh,
        compiler_params=pltpu.CompilerParams(use_tc_tiling_on_sc=False),
        out_type=jax.ShapeDtypeStruct((_BATCH, _SEQ, _EMBED), jnp.float32),
        scratch_types=[
            pltpu.VMEM((_ROWS_PER_IT, _SEQ), jnp.int32),
            pltpu.VMEM((_SEQ, _EMBED), jnp.float32),
        ] + [pltpu.VMEM((_RPC, _SEQ, _EMBED), jnp.float32)] * _NBUF
          + [pltpu.SemaphoreType.DMA] * (2 * _NBUF),
    )
    return f(x, sincos, table)


def kernel(x, table):
    if x.ndim == 1:
        x = x[None, :]
    x = x.astype(jnp.int32)
    freqs = 1.0 / (10000.0 ** (jnp.arange(_HALF, dtype=jnp.float32) / _EMBED))
    ang = jnp.arange(_SEQ, dtype=jnp.float32)[:, None] * freqs[None, :]
    sincos = jnp.concatenate([jnp.cos(ang), jnp.sin(ang)], axis=-1)
    return _sc_call(x, sincos, table)
